# baseline (speedup 1.0000x reference)
import functools

import jax
import jax.numpy as jnp
from jax import lax
from jax.experimental import pallas as pl
from jax.experimental.pallas import tpu as pltpu
from jax.experimental.pallas import tpu_sc as plsc

N = 10000
E = 320000
D = 128
R = 16
EPS = 1e-5

NC = 2
NS = 16
NW = NC * NS
CH = 128
NCHK = 80
SLAB = 8
NSLAB = NCHK // SLAB
EPT = NCHK * CH
EPAD = NW * EPT
NA = N + 8
LANE = 16

BN = 1000
NB = N // BN


def _mm_body(x_ref, w_ref, o_ref):
    o_ref[0] = jnp.dot(x_ref[...], w_ref[0], preferred_element_type=jnp.float32)


def _xw(x, w_all):
    return pl.pallas_call(
        _mm_body,
        grid=(NB, R + 1),
        in_specs=[
            pl.BlockSpec((BN, D), lambda nb, r: (nb, 0)),
            pl.BlockSpec((1, D, D), lambda nb, r: (r, 0, 0)),
        ],
        out_specs=pl.BlockSpec((1, BN, D), lambda nb, r: (r, nb, 0)),
        out_shape=jax.ShapeDtypeStruct((R + 1, N, D), jnp.float32),
    )(x, w_all)


def _make_sc_agg():
    mesh = plsc.VectorSubcoreMesh(core_axis_name="c", subcore_axis_name="s")

    out_type = jax.ShapeDtypeStruct((NC, N, D), jnp.float32)

    scratch = [
        pltpu.VMEM((2, SLAB, CH), jnp.int32),
        pltpu.VMEM((2, SLAB, CH), jnp.int32),
        pltpu.VMEM((2, CH, D), jnp.float32),
        pltpu.SemaphoreType.DMA,
        pltpu.VMEM_SHARED((NA, D), jnp.float32),
    ]

    def body(xw_hbm, gidx_hbm, dst_hbm, z_hbm,
             agg_out, gidxs, dsts, rowb, sem, agg_sh):
        c = lax.axis_index("c")
        s = lax.axis_index("s")
        w = c * NS + s

        @pl.when(s == 0)
        def _():
            pltpu.sync_copy(z_hbm, agg_sh)

        plsc.subcore_barrier()

        pltpu.sync_copy(gidx_hbm.at[w, 0], gidxs.at[0])
        pltpu.sync_copy(dst_hbm.at[w, 0], dsts.at[0])
        pltpu.async_copy(xw_hbm.at[gidxs.at[0, 0]], rowb.at[0], sem)

        def _slab(sl, _):
            cur = lax.rem(sl, 2)
            nxt = lax.rem(sl + 1, 2)

            @pl.when(sl + 1 < NSLAB)
            def _():
                pltpu.sync_copy(gidx_hbm.at[w, sl + 1], gidxs.at[nxt])
                pltpu.sync_copy(dst_hbm.at[w, sl + 1], dsts.at[nxt])

            for j in range(SLAB):
                par = j % 2
                pltpu.make_async_copy(
                    xw_hbm.at[gidxs.at[cur, j]], rowb.at[par], sem).wait()
                if j + 1 < SLAB:
                    pltpu.async_copy(xw_hbm.at[gidxs.at[cur, j + 1]],
                                     rowb.at[1 - par], sem)
                else:
                    @pl.when(sl + 1 < NSLAB)
                    def _():
                        pltpu.async_copy(xw_hbm.at[gidxs.at[nxt, 0]],
                                         rowb.at[1 - par], sem)
                pltpu.sync_copy(rowb.at[par], agg_sh.at[dsts.at[cur, j]],
                                add=True)
            return 0
        lax.fori_loop(0, NSLAB, _slab, 0)

        plsc.subcore_barrier()

        @pl.when(s == 0)
        def _():
            pltpu.sync_copy(agg_sh.at[pl.ds(0, N)], agg_out.at[c])

    return pl.kernel(body, out_type=out_type, mesh=mesh,
                     scratch_types=scratch)


_make_sc_agg = functools.lru_cache(maxsize=None)(_make_sc_agg)


def _sc_agg(*args):
    return _make_sc_agg()(*args)


def _comb_body(hs_ref, a_ref, rd_ref, b_ref, g_ref, be_ref, o_ref):
    a = a_ref[0] + a_ref[1]
    h = hs_ref[...] + a * rd_ref[...] + b_ref[0]
    h = jnp.maximum(h, 0.0)
    mu = jnp.mean(h, axis=1, keepdims=True)
    var = jnp.mean((h - mu) ** 2, axis=1, keepdims=True)
    o_ref[...] = (h - mu) / jnp.sqrt(var + EPS) * g_ref[0] + be_ref[0]


def _combine(hself, agg2, rdegb, b, g, be):
    return pl.pallas_call(
        _comb_body,
        grid=(NB,),
        in_specs=[
            pl.BlockSpec((BN, D), lambda nb: (nb, 0)),
            pl.BlockSpec((NC, BN, D), lambda nb: (0, nb, 0)),
            pl.BlockSpec((BN, D), lambda nb: (nb, 0)),
            pl.BlockSpec((1, D), lambda nb: (0, 0)),
            pl.BlockSpec((1, D), lambda nb: (0, 0)),
            pl.BlockSpec((1, D), lambda nb: (0, 0)),
        ],
        out_specs=pl.BlockSpec((BN, D), lambda nb: (nb, 0)),
        out_shape=jax.ShapeDtypeStruct((N, D), jnp.float32),
    )(hself, agg2, rdegb, b.reshape(1, D), g.reshape(1, D), be.reshape(1, D))


def kernel(node_features, edge_index, edge_types,
           W_rel1, W_self1, b1, g1, be1,
           W_rel2, W_self2, b2, g2, be2):
    gidx = edge_types * N + edge_index[0]
    gidx4 = jnp.concatenate(
        [gidx, jnp.zeros((EPAD - E,), jnp.int32)]).reshape(NW, NSLAB, SLAB, CH)
    dst4 = jnp.concatenate(
        [edge_index[1], jnp.full((EPAD - E,), N, jnp.int32)]
    ).reshape(NW, NSLAB, SLAB, CH)
    zros = jnp.zeros((NA, D), jnp.float32)

    deg = jax.ops.segment_sum(jnp.ones((E,), jnp.float32), edge_index[1],
                              num_segments=N)
    rdegb = jnp.broadcast_to((1.0 / jnp.maximum(deg, 1.0))[:, None], (N, D))

    w_all1 = jnp.concatenate([W_rel1, W_self1[None]], axis=0)
    xw1 = _xw(node_features, w_all1)
    agg1 = _sc_agg(xw1.reshape((R + 1) * N, D), gidx4, dst4, zros)
    h1 = _combine(xw1[R], agg1, rdegb, b1, g1, be1)

    w_all2 = jnp.concatenate([W_rel2, W_self2[None]], axis=0)
    xw2 = _xw(h1, w_all2)
    agg2 = _sc_agg(xw2.reshape((R + 1) * N, D), gidx4, dst4, zros)
    h2 = _combine(xw2[R], agg2, rdegb, b2, g2, be2)
    return h2


# speedup vs baseline: 1.1461x; 1.1461x over previous
import functools

import jax
import jax.numpy as jnp
from jax import lax
from jax.experimental import pallas as pl
from jax.experimental.pallas import tpu as pltpu
from jax.experimental.pallas import tpu_sc as plsc

N = 10000
E = 320000
D = 128
R = 16
EPS = 1e-5

NC = 2
NS = 16
NW = NC * NS
CH = 128
NCHK = 80
SLAB = 8
NSLAB = NCHK // SLAB
EPT = NCHK * CH
RPT = E // NW
PPT = EPT - RPT
NSINK = 512
NA = N + NSINK
LANE = 16

BN = 1000
NB = N // BN


def _mm_body(x_ref, w_ref, o_ref):
    o_ref[0] = jnp.dot(x_ref[...], w_ref[0], preferred_element_type=jnp.float32)


def _xw(x, w_all):
    return pl.pallas_call(
        _mm_body,
        grid=(NB, R + 1),
        in_specs=[
            pl.BlockSpec((BN, D), lambda nb, r: (nb, 0)),
            pl.BlockSpec((1, D, D), lambda nb, r: (r, 0, 0)),
        ],
        out_specs=pl.BlockSpec((1, BN, D), lambda nb, r: (r, nb, 0)),
        out_shape=jax.ShapeDtypeStruct((R + 1, N, D), jnp.float32),
    )(x, w_all)


def _make_sc_agg():
    mesh = plsc.VectorSubcoreMesh(core_axis_name="c", subcore_axis_name="s")

    out_type = jax.ShapeDtypeStruct((NC, N, D), jnp.float32)

    scratch = [
        pltpu.VMEM((2, SLAB, CH), jnp.int32),
        pltpu.VMEM((2, SLAB, CH), jnp.int32),
        pltpu.VMEM((2, CH, D), jnp.float32),
        pltpu.SemaphoreType.DMA,
        pltpu.VMEM_SHARED((NA, D), jnp.float32),
    ]

    def body(xw_hbm, gidx_hbm, dst_hbm, z_hbm,
             agg_out, gidxs, dsts, rowb, sem, agg_sh):
        c = lax.axis_index("c")
        s = lax.axis_index("s")
        w = c * NS + s

        @pl.when(s == 0)
        def _():
            pltpu.sync_copy(z_hbm, agg_sh)

        plsc.subcore_barrier()

        pltpu.sync_copy(gidx_hbm.at[w, 0], gidxs.at[0])
        pltpu.sync_copy(dst_hbm.at[w, 0], dsts.at[0])
        pltpu.async_copy(xw_hbm.at[gidxs.at[0, 0]], rowb.at[0], sem)

        def _slab(sl, _):
            cur = lax.rem(sl, 2)
            nxt = lax.rem(sl + 1, 2)

            @pl.when(sl + 1 < NSLAB)
            def _():
                pltpu.sync_copy(gidx_hbm.at[w, sl + 1], gidxs.at[nxt])
                pltpu.sync_copy(dst_hbm.at[w, sl + 1], dsts.at[nxt])

            for j in range(SLAB):
                par = j % 2
                pltpu.make_async_copy(
                    xw_hbm.at[gidxs.at[cur, j]], rowb.at[par], sem).wait()
                if j + 1 < SLAB:
                    pltpu.async_copy(xw_hbm.at[gidxs.at[cur, j + 1]],
                                     rowb.at[1 - par], sem)
                else:
                    @pl.when(sl + 1 < NSLAB)
                    def _():
                        pltpu.async_copy(xw_hbm.at[gidxs.at[nxt, 0]],
                                         rowb.at[1 - par], sem)
                pltpu.sync_copy(rowb.at[par], agg_sh.at[dsts.at[cur, j]],
                                add=True)
            return 0
        lax.fori_loop(0, NSLAB, _slab, 0)

        plsc.subcore_barrier()

        @pl.when(s == 0)
        def _():
            pltpu.sync_copy(agg_sh.at[pl.ds(0, N)], agg_out.at[c])

    return pl.kernel(body, out_type=out_type, mesh=mesh,
                     scratch_types=scratch)


_make_sc_agg = functools.lru_cache(maxsize=None)(_make_sc_agg)


def _sc_agg(*args):
    return _make_sc_agg()(*args)


def _comb_body(hs_ref, a_ref, rd_ref, b_ref, g_ref, be_ref, o_ref):
    a = a_ref[0] + a_ref[1]
    h = hs_ref[...] + a * rd_ref[...] + b_ref[0]
    h = jnp.maximum(h, 0.0)
    mu = jnp.mean(h, axis=1, keepdims=True)
    var = jnp.mean((h - mu) ** 2, axis=1, keepdims=True)
    o_ref[...] = (h - mu) / jnp.sqrt(var + EPS) * g_ref[0] + be_ref[0]


def _combine(hself, agg2, rdegb, b, g, be):
    return pl.pallas_call(
        _comb_body,
        grid=(NB,),
        in_specs=[
            pl.BlockSpec((BN, D), lambda nb: (nb, 0)),
            pl.BlockSpec((NC, BN, D), lambda nb: (0, nb, 0)),
            pl.BlockSpec((BN, D), lambda nb: (nb, 0)),
            pl.BlockSpec((1, D), lambda nb: (0, 0)),
            pl.BlockSpec((1, D), lambda nb: (0, 0)),
            pl.BlockSpec((1, D), lambda nb: (0, 0)),
        ],
        out_specs=pl.BlockSpec((BN, D), lambda nb: (nb, 0)),
        out_shape=jax.ShapeDtypeStruct((N, D), jnp.float32),
    )(hself, agg2, rdegb, b.reshape(1, D), g.reshape(1, D), be.reshape(1, D))


def kernel(node_features, edge_index, edge_types,
           W_rel1, W_self1, b1, g1, be1,
           W_rel2, W_self2, b2, g2, be2):
    gidx = (edge_types * N + edge_index[0]).reshape(NW, RPT)
    gidx4 = jnp.concatenate(
        [gidx, jnp.zeros((NW, PPT), jnp.int32)], axis=1,
    ).reshape(NW, NSLAB, SLAB, CH)
    dst_pad = jnp.broadcast_to(N + jnp.arange(PPT, dtype=jnp.int32) % NSINK,
                               (NW, PPT))
    dst4 = jnp.concatenate(
        [edge_index[1].reshape(NW, RPT), dst_pad], axis=1,
    ).reshape(NW, NSLAB, SLAB, CH)
    zros = jnp.zeros((NA, D), jnp.float32)

    deg = jax.ops.segment_sum(jnp.ones((E,), jnp.float32), edge_index[1],
                              num_segments=N)
    rdegb = jnp.broadcast_to((1.0 / jnp.maximum(deg, 1.0))[:, None], (N, D))

    w_all1 = jnp.concatenate([W_rel1, W_self1[None]], axis=0)
    xw1 = _xw(node_features, w_all1)
    agg1 = _sc_agg(xw1.reshape((R + 1) * N, D), gidx4, dst4, zros)
    h1 = _combine(xw1[R], agg1, rdegb, b1, g1, be1)

    w_all2 = jnp.concatenate([W_rel2, W_self2[None]], axis=0)
    xw2 = _xw(h1, w_all2)
    agg2 = _sc_agg(xw2.reshape((R + 1) * N, D), gidx4, dst4, zros)
    h2 = _combine(xw2[R], agg2, rdegb, b2, g2, be2)
    return h2


# speedup vs baseline: 1.1462x; 1.0001x over previous
import functools

import jax
import jax.numpy as jnp
from jax import lax
from jax.experimental import pallas as pl
from jax.experimental.pallas import tpu as pltpu
from jax.experimental.pallas import tpu_sc as plsc

N = 10000
E = 320000
D = 128
R = 16
EPS = 1e-5

NC = 2
NS = 16
NW = NC * NS
CH = 128
NCHK = 80
SLAB = 8
NSLAB = NCHK // SLAB
EPT = NCHK * CH
RPT = E // NW
PPT = EPT - RPT
NSINK = 512
NA = N + NSINK
LANE = 16

BN = 1000
NB = N // BN


def _mm_body(x_ref, w_ref, o_ref):
    o_ref[0] = jnp.dot(x_ref[...], w_ref[0], preferred_element_type=jnp.float32)


def _xw(x, w_all):
    return pl.pallas_call(
        _mm_body,
        grid=(NB, R + 1),
        in_specs=[
            pl.BlockSpec((BN, D), lambda nb, r: (nb, 0)),
            pl.BlockSpec((1, D, D), lambda nb, r: (r, 0, 0)),
        ],
        out_specs=pl.BlockSpec((1, BN, D), lambda nb, r: (r, nb, 0)),
        out_shape=jax.ShapeDtypeStruct((R + 1, N, D), jnp.float32),
    )(x, w_all)


def _make_sc_agg():
    mesh = plsc.VectorSubcoreMesh(core_axis_name="c", subcore_axis_name="s")

    out_type = jax.ShapeDtypeStruct((NC, N, D), jnp.float32)

    scratch = [
        pltpu.VMEM((2, SLAB, CH), jnp.int32),
        pltpu.VMEM((2, SLAB, CH), jnp.int32),
        pltpu.VMEM((2, CH, D), jnp.float32),
        pltpu.SemaphoreType.DMA,
        pltpu.SemaphoreType.DMA,
        pltpu.VMEM_SHARED((NA, D), jnp.float32),
    ]

    def body(xw_hbm, gidx_hbm, dst_hbm, z_hbm,
             agg_out, gidxs, dsts, rowb, sem, sem_s, agg_sh):
        c = lax.axis_index("c")
        s = lax.axis_index("s")
        w = c * NS + s

        @pl.when(s == 0)
        def _():
            pltpu.sync_copy(z_hbm, agg_sh)

        plsc.subcore_barrier()

        pltpu.sync_copy(gidx_hbm.at[w, 0], gidxs.at[0])
        pltpu.sync_copy(dst_hbm.at[w, 0], dsts.at[0])
        pltpu.async_copy(xw_hbm.at[gidxs.at[0, 0]], rowb.at[0], sem)

        def _slab(sl, _):
            cur = lax.rem(sl, 2)
            nxt = lax.rem(sl + 1, 2)

            @pl.when(sl > 0)
            def _():
                pltpu.make_async_copy(
                    rowb.at[1], agg_sh.at[dsts.at[nxt, SLAB - 1]],
                    sem_s).wait()

            @pl.when(sl + 1 < NSLAB)
            def _():
                pltpu.sync_copy(gidx_hbm.at[w, sl + 1], gidxs.at[nxt])
                pltpu.sync_copy(dst_hbm.at[w, sl + 1], dsts.at[nxt])

            for j in range(SLAB):
                par = j % 2
                pltpu.make_async_copy(
                    xw_hbm.at[gidxs.at[cur, j]], rowb.at[par], sem).wait()
                if j >= 1:
                    pltpu.make_async_copy(
                        rowb.at[1 - par], agg_sh.at[dsts.at[cur, j - 1]],
                        sem_s).wait()
                pltpu.async_copy(rowb.at[par], agg_sh.at[dsts.at[cur, j]],
                                 sem_s, add=True)
                if j + 1 < SLAB:
                    pltpu.async_copy(xw_hbm.at[gidxs.at[cur, j + 1]],
                                     rowb.at[1 - par], sem)
                else:
                    @pl.when(sl + 1 < NSLAB)
                    def _():
                        pltpu.async_copy(xw_hbm.at[gidxs.at[nxt, 0]],
                                         rowb.at[1 - par], sem)
            return 0
        lax.fori_loop(0, NSLAB, _slab, 0)

        pltpu.make_async_copy(
            rowb.at[1], agg_sh.at[dsts.at[(NSLAB - 1) % 2, SLAB - 1]],
            sem_s).wait()

        plsc.subcore_barrier()

        @pl.when(s == 0)
        def _():
            pltpu.sync_copy(agg_sh.at[pl.ds(0, N)], agg_out.at[c])

    return pl.kernel(body, out_type=out_type, mesh=mesh,
                     scratch_types=scratch)


_make_sc_agg = functools.lru_cache(maxsize=None)(_make_sc_agg)


def _sc_agg(*args):
    return _make_sc_agg()(*args)


def _comb_body(hs_ref, a_ref, rd_ref, b_ref, g_ref, be_ref, o_ref):
    a = a_ref[0] + a_ref[1]
    h = hs_ref[...] + a * rd_ref[...] + b_ref[0]
    h = jnp.maximum(h, 0.0)
    mu = jnp.mean(h, axis=1, keepdims=True)
    var = jnp.mean((h - mu) ** 2, axis=1, keepdims=True)
    o_ref[...] = (h - mu) / jnp.sqrt(var + EPS) * g_ref[0] + be_ref[0]


def _combine(hself, agg2, rdegb, b, g, be):
    return pl.pallas_call(
        _comb_body,
        grid=(NB,),
        in_specs=[
            pl.BlockSpec((BN, D), lambda nb: (nb, 0)),
            pl.BlockSpec((NC, BN, D), lambda nb: (0, nb, 0)),
            pl.BlockSpec((BN, D), lambda nb: (nb, 0)),
            pl.BlockSpec((1, D), lambda nb: (0, 0)),
            pl.BlockSpec((1, D), lambda nb: (0, 0)),
            pl.BlockSpec((1, D), lambda nb: (0, 0)),
        ],
        out_specs=pl.BlockSpec((BN, D), lambda nb: (nb, 0)),
        out_shape=jax.ShapeDtypeStruct((N, D), jnp.float32),
    )(hself, agg2, rdegb, b.reshape(1, D), g.reshape(1, D), be.reshape(1, D))


def kernel(node_features, edge_index, edge_types,
           W_rel1, W_self1, b1, g1, be1,
           W_rel2, W_self2, b2, g2, be2):
    gidx = (edge_types * N + edge_index[0]).reshape(NW, RPT)
    gidx4 = jnp.concatenate(
        [gidx, jnp.zeros((NW, PPT), jnp.int32)], axis=1,
    ).reshape(NW, NSLAB, SLAB, CH)
    dst_pad = jnp.broadcast_to(N + jnp.arange(PPT, dtype=jnp.int32) % NSINK,
                               (NW, PPT))
    dst4 = jnp.concatenate(
        [edge_index[1].reshape(NW, RPT), dst_pad], axis=1,
    ).reshape(NW, NSLAB, SLAB, CH)
    zros = jnp.zeros((NA, D), jnp.float32)

    deg = jax.ops.segment_sum(jnp.ones((E,), jnp.float32), edge_index[1],
                              num_segments=N)
    rdegb = jnp.broadcast_to((1.0 / jnp.maximum(deg, 1.0))[:, None], (N, D))

    w_all1 = jnp.concatenate([W_rel1, W_self1[None]], axis=0)
    xw1 = _xw(node_features, w_all1)
    agg1 = _sc_agg(xw1.reshape((R + 1) * N, D), gidx4, dst4, zros)
    h1 = _combine(xw1[R], agg1, rdegb, b1, g1, be1)

    w_all2 = jnp.concatenate([W_rel2, W_self2[None]], axis=0)
    xw2 = _xw(h1, w_all2)
    agg2 = _sc_agg(xw2.reshape((R + 1) * N, D), gidx4, dst4, zros)
    h2 = _combine(xw2[R], agg2, rdegb, b2, g2, be2)
    return h2


# speedup vs baseline: 1.8381x; 1.6036x over previous
import functools

import jax
import jax.numpy as jnp
from jax import lax
from jax.experimental import pallas as pl
from jax.experimental.pallas import tpu as pltpu
from jax.experimental.pallas import tpu_sc as plsc

N = 10000
E = 320000
D = 128
R = 16
EPS = 1e-5

NC = 2
NS = 16
NW = NC * NS
CH = 125
NCHK = 80
SLAB = 8
NSLAB = NCHK // SLAB
NA = N
LANE = 16

BN = 1000
NB = N // BN


def _mm_body(x_ref, w_ref, o_ref):
    o_ref[0] = jnp.dot(x_ref[...], w_ref[0], preferred_element_type=jnp.float32)


def _xw(x, w_all):
    return pl.pallas_call(
        _mm_body,
        grid=(NB, R + 1),
        in_specs=[
            pl.BlockSpec((BN, D), lambda nb, r: (nb, 0)),
            pl.BlockSpec((1, D, D), lambda nb, r: (r, 0, 0)),
        ],
        out_specs=pl.BlockSpec((1, BN, D), lambda nb, r: (r, nb, 0)),
        out_shape=jax.ShapeDtypeStruct((R + 1, N, D), jnp.float32),
    )(x, w_all)


def _make_sc_agg():
    mesh = plsc.VectorSubcoreMesh(core_axis_name="c", subcore_axis_name="s")

    out_type = jax.ShapeDtypeStruct((NC, N, D), jnp.float32)

    scratch = [
        pltpu.VMEM((2, SLAB, CH), jnp.int32),
        pltpu.VMEM((2, SLAB, CH), jnp.int32),
        pltpu.VMEM((2, CH, D), jnp.float32),
        pltpu.SemaphoreType.DMA,
        pltpu.SemaphoreType.DMA,
        pltpu.VMEM_SHARED((NA, D), jnp.float32),
    ]

    def body(xw_hbm, gidx_hbm, dst_hbm, z_hbm,
             agg_out, gidxs, dsts, rowb, sem, sem_s, agg_sh):
        c = lax.axis_index("c")
        s = lax.axis_index("s")
        w = c * NS + s

        @pl.when(s == 0)
        def _():
            pltpu.sync_copy(z_hbm, agg_sh)

        plsc.subcore_barrier()

        pltpu.sync_copy(gidx_hbm.at[w, 0], gidxs.at[0])
        pltpu.sync_copy(dst_hbm.at[w, 0], dsts.at[0])
        pltpu.async_copy(xw_hbm.at[gidxs.at[0, 0]], rowb.at[0], sem)

        def _slab(sl, _):
            cur = lax.rem(sl, 2)
            nxt = lax.rem(sl + 1, 2)

            @pl.when(sl > 0)
            def _():
                pltpu.make_async_copy(
                    rowb.at[1], agg_sh.at[dsts.at[nxt, SLAB - 1]],
                    sem_s).wait()

            @pl.when(sl + 1 < NSLAB)
            def _():
                pltpu.sync_copy(gidx_hbm.at[w, sl + 1], gidxs.at[nxt])
                pltpu.sync_copy(dst_hbm.at[w, sl + 1], dsts.at[nxt])

            for j in range(SLAB):
                par = j % 2
                pltpu.make_async_copy(
                    xw_hbm.at[gidxs.at[cur, j]], rowb.at[par], sem).wait()
                if j >= 1:
                    pltpu.make_async_copy(
                        rowb.at[1 - par], agg_sh.at[dsts.at[cur, j - 1]],
                        sem_s).wait()
                pltpu.async_copy(rowb.at[par], agg_sh.at[dsts.at[cur, j]],
                                 sem_s, add=True)
                if j + 1 < SLAB:
                    pltpu.async_copy(xw_hbm.at[gidxs.at[cur, j + 1]],
                                     rowb.at[1 - par], sem)
                else:
                    @pl.when(sl + 1 < NSLAB)
                    def _():
                        pltpu.async_copy(xw_hbm.at[gidxs.at[nxt, 0]],
                                         rowb.at[1 - par], sem)
            return 0
        lax.fori_loop(0, NSLAB, _slab, 0)

        pltpu.make_async_copy(
            rowb.at[1], agg_sh.at[dsts.at[(NSLAB - 1) % 2, SLAB - 1]],
            sem_s).wait()

        plsc.subcore_barrier()

        @pl.when(s == 0)
        def _():
            pltpu.sync_copy(agg_sh.at[pl.ds(0, N)], agg_out.at[c])

    return pl.kernel(body, out_type=out_type, mesh=mesh,
                     scratch_types=scratch)


_make_sc_agg = functools.lru_cache(maxsize=None)(_make_sc_agg)


def _sc_agg(*args):
    return _make_sc_agg()(*args)


def _comb_body(hs_ref, a_ref, rd_ref, b_ref, g_ref, be_ref, o_ref):
    a = a_ref[0] + a_ref[1]
    h = hs_ref[...] + a * rd_ref[...] + b_ref[0]
    h = jnp.maximum(h, 0.0)
    mu = jnp.mean(h, axis=1, keepdims=True)
    var = jnp.mean((h - mu) ** 2, axis=1, keepdims=True)
    o_ref[...] = (h - mu) / jnp.sqrt(var + EPS) * g_ref[0] + be_ref[0]


def _combine(hself, agg2, rdegb, b, g, be):
    return pl.pallas_call(
        _comb_body,
        grid=(NB,),
        in_specs=[
            pl.BlockSpec((BN, D), lambda nb: (nb, 0)),
            pl.BlockSpec((NC, BN, D), lambda nb: (0, nb, 0)),
            pl.BlockSpec((BN, D), lambda nb: (nb, 0)),
            pl.BlockSpec((1, D), lambda nb: (0, 0)),
            pl.BlockSpec((1, D), lambda nb: (0, 0)),
            pl.BlockSpec((1, D), lambda nb: (0, 0)),
        ],
        out_specs=pl.BlockSpec((BN, D), lambda nb: (nb, 0)),
        out_shape=jax.ShapeDtypeStruct((N, D), jnp.float32),
    )(hself, agg2, rdegb, b.reshape(1, D), g.reshape(1, D), be.reshape(1, D))


def kernel(node_features, edge_index, edge_types,
           W_rel1, W_self1, b1, g1, be1,
           W_rel2, W_self2, b2, g2, be2):
    gidx4 = (edge_types * N + edge_index[0]).reshape(NW, NSLAB, SLAB, CH)
    dst4 = edge_index[1].reshape(NW, NSLAB, SLAB, CH)
    zros = jnp.zeros((NA, D), jnp.float32)

    deg = jax.ops.segment_sum(jnp.ones((E,), jnp.float32), edge_index[1],
                              num_segments=N)
    rdegb = jnp.broadcast_to((1.0 / jnp.maximum(deg, 1.0))[:, None], (N, D))

    w_all1 = jnp.concatenate([W_rel1, W_self1[None]], axis=0)
    xw1 = _xw(node_features, w_all1)
    agg1 = _sc_agg(xw1.reshape((R + 1) * N, D), gidx4, dst4, zros)
    h1 = _combine(xw1[R], agg1, rdegb, b1, g1, be1)

    w_all2 = jnp.concatenate([W_rel2, W_self2[None]], axis=0)
    xw2 = _xw(h1, w_all2)
    agg2 = _sc_agg(xw2.reshape((R + 1) * N, D), gidx4, dst4, zros)
    h2 = _combine(xw2[R], agg2, rdegb, b2, g2, be2)
    return h2


# speedup vs baseline: 2.6478x; 1.4405x over previous
import functools

import jax
import jax.numpy as jnp
from jax import lax
from jax.experimental import pallas as pl
from jax.experimental.pallas import tpu as pltpu
from jax.experimental.pallas import tpu_sc as plsc

N = 10000
E = 320000
D = 128
R = 16
EPS = 1e-5

NC = 2
NS = 16
NW = NC * NS
CH = 125
NCHK = 80
SLAB = 8
NSLAB = NCHK // SLAB
NA = N
LANE = 16

BN = 1000
NB = N // BN


def _mm_body(x_ref, w_ref, o_ref):
    o_ref[0] = jnp.dot(x_ref[...], w_ref[0], preferred_element_type=jnp.float32)


def _xw(x, w_all):
    return pl.pallas_call(
        _mm_body,
        grid=(NB, R + 1),
        in_specs=[
            pl.BlockSpec((BN, D), lambda nb, r: (nb, 0)),
            pl.BlockSpec((1, D, D), lambda nb, r: (r, 0, 0)),
        ],
        out_specs=pl.BlockSpec((1, BN, D), lambda nb, r: (r, nb, 0)),
        out_shape=jax.ShapeDtypeStruct((R + 1, N, D), jnp.float32),
    )(x, w_all)


def _make_sc_agg():
    mesh = plsc.VectorSubcoreMesh(core_axis_name="c", subcore_axis_name="s")

    out_type = jax.ShapeDtypeStruct((NC, N, D), jnp.float32)

    scratch = [
        pltpu.VMEM((2, SLAB, CH), jnp.int32),
        pltpu.VMEM((2, SLAB, CH), jnp.int32),
        pltpu.VMEM((2, CH, D), jnp.float32),
        pltpu.SemaphoreType.DMA,
        pltpu.SemaphoreType.DMA,
        pltpu.VMEM_SHARED((NA, D), jnp.float32),
    ]

    def body(xw_hbm, gidx_hbm, dst_hbm, z_hbm,
             agg_out, gidxs, dsts, rowb, sem, sem_s, agg_sh):
        c = lax.axis_index("c")
        s = lax.axis_index("s")
        w = c * NS + s

        @pl.when(s == 0)
        def _():
            pltpu.sync_copy(z_hbm, agg_sh)

        plsc.subcore_barrier()

        pltpu.sync_copy(gidx_hbm.at[w, 0], gidxs.at[0])
        pltpu.sync_copy(dst_hbm.at[w, 0], dsts.at[0])
        pltpu.async_copy(xw_hbm.at[gidxs.at[0, 0]], rowb.at[0], sem)

        def _slab(sl, _):
            cur = lax.rem(sl, 2)
            nxt = lax.rem(sl + 1, 2)

            @pl.when(sl > 0)
            def _():
                pltpu.make_async_copy(
                    rowb.at[1], agg_sh.at[dsts.at[nxt, SLAB - 1]],
                    sem_s).wait()

            @pl.when(sl + 1 < NSLAB)
            def _():
                pltpu.sync_copy(gidx_hbm.at[w, sl + 1], gidxs.at[nxt])
                pltpu.sync_copy(dst_hbm.at[w, sl + 1], dsts.at[nxt])

            for j in range(SLAB):
                par = j % 2
                pltpu.make_async_copy(
                    xw_hbm.at[gidxs.at[cur, j]], rowb.at[par], sem).wait()
                if j >= 1:
                    pltpu.make_async_copy(
                        rowb.at[1 - par], agg_sh.at[dsts.at[cur, j - 1]],
                        sem_s).wait()
                pltpu.async_copy(rowb.at[par], agg_sh.at[dsts.at[cur, j]],
                                 sem_s, add=True)
                if j + 1 < SLAB:
                    pltpu.async_copy(xw_hbm.at[gidxs.at[cur, j + 1]],
                                     rowb.at[1 - par], sem)
                else:
                    @pl.when(sl + 1 < NSLAB)
                    def _():
                        pltpu.async_copy(xw_hbm.at[gidxs.at[nxt, 0]],
                                         rowb.at[1 - par], sem)
            return 0
        lax.fori_loop(0, NSLAB, _slab, 0)

        pltpu.make_async_copy(
            rowb.at[1], agg_sh.at[dsts.at[(NSLAB - 1) % 2, SLAB - 1]],
            sem_s).wait()

        plsc.subcore_barrier()

        @pl.when(s == 0)
        def _():
            pltpu.sync_copy(agg_sh.at[pl.ds(0, N)], agg_out.at[c])

    return pl.kernel(body, out_type=out_type, mesh=mesh,
                     scratch_types=scratch)


_make_sc_agg = functools.lru_cache(maxsize=None)(_make_sc_agg)


def _sc_agg(*args):
    return _make_sc_agg()(*args)


def _make_sc_deg():
    mesh = plsc.VectorSubcoreMesh(core_axis_name="c", subcore_axis_name="s")

    out_type = jax.ShapeDtypeStruct((NC, N, D), jnp.float32)

    scratch = [
        pltpu.VMEM((2, SLAB, CH), jnp.int32),
        pltpu.VMEM((CH, D), jnp.float32),
        pltpu.SemaphoreType.DMA,
        pltpu.VMEM_SHARED((N, D), jnp.float32),
    ]

    def body(dst_hbm, z_hbm, ones_hbm, deg_out, dsts, onesb, sem_s, deg_sh):
        c = lax.axis_index("c")
        s = lax.axis_index("s")
        w = c * NS + s

        @pl.when(s == 0)
        def _():
            pltpu.sync_copy(z_hbm, deg_sh)
        pltpu.sync_copy(ones_hbm, onesb)

        plsc.subcore_barrier()

        pltpu.sync_copy(dst_hbm.at[w, 0], dsts.at[0])

        def _slab(sl, _):
            cur = lax.rem(sl, 2)
            nxt = lax.rem(sl + 1, 2)

            @pl.when(sl > 0)
            def _():
                for j in range(SLAB):
                    pltpu.make_async_copy(
                        onesb, deg_sh.at[dsts.at[nxt, j]], sem_s).wait()

            @pl.when(sl + 1 < NSLAB)
            def _():
                pltpu.sync_copy(dst_hbm.at[w, sl + 1], dsts.at[nxt])

            for j in range(SLAB):
                pltpu.async_copy(onesb, deg_sh.at[dsts.at[cur, j]],
                                 sem_s, add=True)
            return 0
        lax.fori_loop(0, NSLAB, _slab, 0)

        for j in range(SLAB):
            pltpu.make_async_copy(
                onesb, deg_sh.at[dsts.at[(NSLAB - 1) % 2, j]], sem_s).wait()

        plsc.subcore_barrier()

        @pl.when(s == 0)
        def _():
            pltpu.sync_copy(deg_sh, deg_out.at[c])

    return pl.kernel(body, out_type=out_type, mesh=mesh,
                     scratch_types=scratch)


_make_sc_deg = functools.lru_cache(maxsize=None)(_make_sc_deg)


def _sc_deg(*args):
    return _make_sc_deg()(*args)


def _comb_body(hs_ref, a_ref, d_ref, b_ref, g_ref, be_ref, o_ref):
    a = a_ref[0] + a_ref[1]
    dsum = d_ref[0, :, 0] + d_ref[1, :, 0]
    rd = 1.0 / jnp.maximum(dsum, 1.0)
    h = hs_ref[...] + a * rd[:, None] + b_ref[0]
    h = jnp.maximum(h, 0.0)
    mu = jnp.mean(h, axis=1, keepdims=True)
    var = jnp.mean((h - mu) ** 2, axis=1, keepdims=True)
    o_ref[...] = (h - mu) / jnp.sqrt(var + EPS) * g_ref[0] + be_ref[0]


def _combine(hself, agg2, degs, b, g, be):
    return pl.pallas_call(
        _comb_body,
        grid=(NB,),
        in_specs=[
            pl.BlockSpec((BN, D), lambda nb: (nb, 0)),
            pl.BlockSpec((NC, BN, D), lambda nb: (0, nb, 0)),
            pl.BlockSpec((NC, BN, D), lambda nb: (0, nb, 0)),
            pl.BlockSpec((1, D), lambda nb: (0, 0)),
            pl.BlockSpec((1, D), lambda nb: (0, 0)),
            pl.BlockSpec((1, D), lambda nb: (0, 0)),
        ],
        out_specs=pl.BlockSpec((BN, D), lambda nb: (nb, 0)),
        out_shape=jax.ShapeDtypeStruct((N, D), jnp.float32),
    )(hself, agg2, degs, b.reshape(1, D), g.reshape(1, D), be.reshape(1, D))


def kernel(node_features, edge_index, edge_types,
           W_rel1, W_self1, b1, g1, be1,
           W_rel2, W_self2, b2, g2, be2):
    gidx4 = (edge_types * N + edge_index[0]).reshape(NW, NSLAB, SLAB, CH)
    dst4 = edge_index[1].reshape(NW, NSLAB, SLAB, CH)
    zros = jnp.zeros((NA, D), jnp.float32)
    ones = jnp.ones((CH, D), jnp.float32)

    degs = _sc_deg(dst4, zros, ones)

    w_all1 = jnp.concatenate([W_rel1, W_self1[None]], axis=0)
    xw1 = _xw(node_features, w_all1)
    agg1 = _sc_agg(xw1.reshape((R + 1) * N, D), gidx4, dst4, zros)
    h1 = _combine(xw1[R], agg1, degs, b1, g1, be1)

    w_all2 = jnp.concatenate([W_rel2, W_self2[None]], axis=0)
    xw2 = _xw(h1, w_all2)
    agg2 = _sc_agg(xw2.reshape((R + 1) * N, D), gidx4, dst4, zros)
    h2 = _combine(xw2[R], agg2, degs, b2, g2, be2)
    return h2


# speedup vs baseline: 3.1215x; 1.1789x over previous
import functools

import jax
import jax.numpy as jnp
from jax import lax
from jax.experimental import pallas as pl
from jax.experimental.pallas import tpu as pltpu
from jax.experimental.pallas import tpu_sc as plsc

N = 10000
E = 320000
D = 128
R = 16
EPS = 1e-5

NC = 2
NS = 16
NW = NC * NS
CH = 125
NCHK = 80
SLAB = 8
NSLAB = NCHK // SLAB
NA = N
LANE = 16

BN = 2000
NB = N // BN
BNC = 1000
NBC = N // BNC


def _mm_body(x_ref, w_ref, o_ref):
    o_ref[0] = jnp.dot(x_ref[...], w_ref[0], preferred_element_type=jnp.float32)


def _xw(x, w_all):
    return pl.pallas_call(
        _mm_body,
        grid=(NB, R + 1),
        in_specs=[
            pl.BlockSpec((BN, D), lambda nb, r: (nb, 0)),
            pl.BlockSpec((1, D, D), lambda nb, r: (r, 0, 0)),
        ],
        out_specs=pl.BlockSpec((1, BN, D), lambda nb, r: (r, nb, 0)),
        out_shape=jax.ShapeDtypeStruct((R + 1, N, D), jnp.float32),
    )(x, w_all)


def _make_sc_agg():
    mesh = plsc.VectorSubcoreMesh(core_axis_name="c", subcore_axis_name="s")

    out_type = jax.ShapeDtypeStruct((NC, N, D), jnp.float32)

    scratch = [
        pltpu.VMEM((2, SLAB, CH), jnp.int32),
        pltpu.VMEM((2, SLAB, CH), jnp.int32),
        pltpu.VMEM((2, CH, D), jnp.float32),
        pltpu.SemaphoreType.DMA,
        pltpu.SemaphoreType.DMA,
        pltpu.VMEM_SHARED((NA, D), jnp.float32),
    ]

    def body(xw_hbm, gidx_hbm, dst_hbm, z_hbm,
             agg_out, gidxs, dsts, rowb, sem, sem_s, agg_sh):
        c = lax.axis_index("c")
        s = lax.axis_index("s")
        w = c * NS + s

        @pl.when(s == 0)
        def _():
            pltpu.sync_copy(z_hbm, agg_sh)

        plsc.subcore_barrier()

        pltpu.sync_copy(gidx_hbm.at[w, 0], gidxs.at[0])
        pltpu.sync_copy(dst_hbm.at[w, 0], dsts.at[0])
        pltpu.async_copy(xw_hbm.at[gidxs.at[0, 0]], rowb.at[0], sem)

        def _slab(sl, _):
            cur = lax.rem(sl, 2)
            nxt = lax.rem(sl + 1, 2)

            @pl.when(sl > 0)
            def _():
                pltpu.make_async_copy(
                    rowb.at[1], agg_sh.at[dsts.at[nxt, SLAB - 1]],
                    sem_s).wait()

            @pl.when(sl + 1 < NSLAB)
            def _():
                pltpu.sync_copy(gidx_hbm.at[w, sl + 1], gidxs.at[nxt])
                pltpu.sync_copy(dst_hbm.at[w, sl + 1], dsts.at[nxt])

            for j in range(SLAB):
                par = j % 2
                pltpu.make_async_copy(
                    xw_hbm.at[gidxs.at[cur, j]], rowb.at[par], sem).wait()
                if j >= 1:
                    pltpu.make_async_copy(
                        rowb.at[1 - par], agg_sh.at[dsts.at[cur, j - 1]],
                        sem_s).wait()
                pltpu.async_copy(rowb.at[par], agg_sh.at[dsts.at[cur, j]],
                                 sem_s, add=True)
                if j + 1 < SLAB:
                    pltpu.async_copy(xw_hbm.at[gidxs.at[cur, j + 1]],
                                     rowb.at[1 - par], sem)
                else:
                    @pl.when(sl + 1 < NSLAB)
                    def _():
                        pltpu.async_copy(xw_hbm.at[gidxs.at[nxt, 0]],
                                         rowb.at[1 - par], sem)
            return 0
        lax.fori_loop(0, NSLAB, _slab, 0)

        pltpu.make_async_copy(
            rowb.at[1], agg_sh.at[dsts.at[(NSLAB - 1) % 2, SLAB - 1]],
            sem_s).wait()

        plsc.subcore_barrier()

        @pl.when(s == 0)
        def _():
            pltpu.sync_copy(agg_sh.at[pl.ds(0, N)], agg_out.at[c])

    return pl.kernel(body, out_type=out_type, mesh=mesh,
                     scratch_types=scratch)


_make_sc_agg = functools.lru_cache(maxsize=None)(_make_sc_agg)


def _sc_agg(*args):
    return _make_sc_agg()(*args)


def _make_sc_deg():
    mesh = plsc.VectorSubcoreMesh(core_axis_name="c", subcore_axis_name="s")

    out_type = jax.ShapeDtypeStruct((NC, N, D), jnp.float32)

    scratch = [
        pltpu.VMEM((2, SLAB, CH), jnp.int32),
        pltpu.VMEM((CH, D), jnp.float32),
        pltpu.SemaphoreType.DMA,
        pltpu.VMEM_SHARED((N, D), jnp.float32),
    ]

    def body(dst_hbm, z_hbm, ones_hbm, deg_out, dsts, onesb, sem_s, deg_sh):
        c = lax.axis_index("c")
        s = lax.axis_index("s")
        w = c * NS + s

        @pl.when(s == 0)
        def _():
            pltpu.sync_copy(z_hbm, deg_sh)
        pltpu.sync_copy(ones_hbm, onesb)

        plsc.subcore_barrier()

        pltpu.sync_copy(dst_hbm.at[w, 0], dsts.at[0])

        def _slab(sl, _):
            cur = lax.rem(sl, 2)
            nxt = lax.rem(sl + 1, 2)

            @pl.when(sl > 0)
            def _():
                for j in range(SLAB):
                    pltpu.make_async_copy(
                        onesb, deg_sh.at[dsts.at[nxt, j]], sem_s).wait()

            @pl.when(sl + 1 < NSLAB)
            def _():
                pltpu.sync_copy(dst_hbm.at[w, sl + 1], dsts.at[nxt])

            for j in range(SLAB):
                pltpu.async_copy(onesb, deg_sh.at[dsts.at[cur, j]],
                                 sem_s, add=True)
            return 0
        lax.fori_loop(0, NSLAB, _slab, 0)

        for j in range(SLAB):
            pltpu.make_async_copy(
                onesb, deg_sh.at[dsts.at[(NSLAB - 1) % 2, j]], sem_s).wait()

        plsc.subcore_barrier()

        @pl.when(s == 0)
        def _():
            pltpu.sync_copy(deg_sh, deg_out.at[c])

    return pl.kernel(body, out_type=out_type, mesh=mesh,
                     scratch_types=scratch)


_make_sc_deg = functools.lru_cache(maxsize=None)(_make_sc_deg)


def _sc_deg(*args):
    return _make_sc_deg()(*args)


def _comb_body(hs_ref, a_ref, d_ref, b_ref, g_ref, be_ref, o_ref):
    a = a_ref[0] + a_ref[1]
    dsum = d_ref[0, :, 0] + d_ref[1, :, 0]
    rd = 1.0 / jnp.maximum(dsum, 1.0)
    h = hs_ref[...] + a * rd[:, None] + b_ref[0]
    h = jnp.maximum(h, 0.0)
    mu = jnp.mean(h, axis=1, keepdims=True)
    var = jnp.mean((h - mu) ** 2, axis=1, keepdims=True)
    o_ref[...] = (h - mu) / jnp.sqrt(var + EPS) * g_ref[0] + be_ref[0]


def _combine(hself, agg2, degs, b, g, be):
    return pl.pallas_call(
        _comb_body,
        grid=(NBC,),
        in_specs=[
            pl.BlockSpec((BNC, D), lambda nb: (nb, 0)),
            pl.BlockSpec((NC, BNC, D), lambda nb: (0, nb, 0)),
            pl.BlockSpec((NC, BNC, D), lambda nb: (0, nb, 0)),
            pl.BlockSpec((1, D), lambda nb: (0, 0)),
            pl.BlockSpec((1, D), lambda nb: (0, 0)),
            pl.BlockSpec((1, D), lambda nb: (0, 0)),
        ],
        out_specs=pl.BlockSpec((BNC, D), lambda nb: (nb, 0)),
        out_shape=jax.ShapeDtypeStruct((N, D), jnp.float32),
    )(hself, agg2, degs, b.reshape(1, D), g.reshape(1, D), be.reshape(1, D))


def kernel(node_features, edge_index, edge_types,
           W_rel1, W_self1, b1, g1, be1,
           W_rel2, W_self2, b2, g2, be2):
    gidx4 = (edge_types * N + edge_index[0]).reshape(NW, NSLAB, SLAB, CH)
    dst4 = edge_index[1].reshape(NW, NSLAB, SLAB, CH)
    zros = jnp.zeros((NA, D), jnp.float32)
    ones = jnp.ones((CH, D), jnp.float32)

    degs = _sc_deg(dst4, zros, ones)

    w_all1 = jnp.concatenate([W_rel1, W_self1[None]], axis=0)
    xw1 = _xw(node_features, w_all1)
    agg1 = _sc_agg(xw1.reshape((R + 1) * N, D), gidx4, dst4, zros)
    h1 = _combine(xw1[R], agg1, degs, b1, g1, be1)

    w_all2 = jnp.concatenate([W_rel2, W_self2[None]], axis=0)
    xw2 = _xw(h1, w_all2)
    agg2 = _sc_agg(xw2.reshape((R + 1) * N, D), gidx4, dst4, zros)
    h2 = _combine(xw2[R], agg2, degs, b2, g2, be2)
    return h2


# speedup vs baseline: 3.6489x; 1.1690x over previous
import functools

import jax
import jax.numpy as jnp
from jax import lax
from jax.experimental import pallas as pl
from jax.experimental.pallas import tpu as pltpu
from jax.experimental.pallas import tpu_sc as plsc

N = 10000
E = 320000
D = 128
R = 16
EPS = 1e-5

NC = 2
NS = 16
NW = NC * NS
CH = 125
NCHK = 80
SLAB = 8
NSLAB = NCHK // SLAB
NA = N
LANE = 16

BN = 10000
NB = N // BN
BNC = 1000
NBC = N // BNC


def _mm_body(x_ref, w_ref, o_ref):
    o_ref[0] = jnp.dot(x_ref[...], w_ref[0], preferred_element_type=jnp.float32)


def _xw(x, w_all):
    return pl.pallas_call(
        _mm_body,
        grid=(NB, R + 1),
        in_specs=[
            pl.BlockSpec((BN, D), lambda nb, r: (nb, 0)),
            pl.BlockSpec((1, D, D), lambda nb, r: (r, 0, 0)),
        ],
        out_specs=pl.BlockSpec((1, BN, D), lambda nb, r: (r, nb, 0)),
        out_shape=jax.ShapeDtypeStruct((R + 1, N, D), jnp.float32),
    )(x, w_all)


def _make_sc_agg():
    mesh = plsc.VectorSubcoreMesh(core_axis_name="c", subcore_axis_name="s")

    out_type = jax.ShapeDtypeStruct((NC, N, D), jnp.float32)

    scratch = [
        pltpu.VMEM((2, SLAB, CH), jnp.int32),
        pltpu.VMEM((2, SLAB, CH), jnp.int32),
        pltpu.VMEM((2, CH, D), jnp.float32),
        pltpu.SemaphoreType.DMA,
        pltpu.SemaphoreType.DMA,
        pltpu.VMEM_SHARED((NA, D), jnp.float32),
    ]

    def body(xw_hbm, gidx_hbm, dst_hbm, z_hbm,
             agg_out, gidxs, dsts, rowb, sem, sem_s, agg_sh):
        c = lax.axis_index("c")
        s = lax.axis_index("s")
        w = c * NS + s

        @pl.when(s == 0)
        def _():
            pltpu.sync_copy(z_hbm, agg_sh)

        plsc.subcore_barrier()

        pltpu.sync_copy(gidx_hbm.at[w, 0], gidxs.at[0])
        pltpu.sync_copy(dst_hbm.at[w, 0], dsts.at[0])
        pltpu.async_copy(xw_hbm.at[gidxs.at[0, 0]], rowb.at[0], sem)

        def _slab(sl, _):
            cur = lax.rem(sl, 2)
            nxt = lax.rem(sl + 1, 2)

            @pl.when(sl > 0)
            def _():
                pltpu.make_async_copy(
                    rowb.at[1], agg_sh.at[dsts.at[nxt, SLAB - 1]],
                    sem_s).wait()

            @pl.when(sl + 1 < NSLAB)
            def _():
                pltpu.sync_copy(gidx_hbm.at[w, sl + 1], gidxs.at[nxt])
                pltpu.sync_copy(dst_hbm.at[w, sl + 1], dsts.at[nxt])

            for j in range(SLAB):
                par = j % 2
                pltpu.make_async_copy(
                    xw_hbm.at[gidxs.at[cur, j]], rowb.at[par], sem).wait()
                if j >= 1:
                    pltpu.make_async_copy(
                        rowb.at[1 - par], agg_sh.at[dsts.at[cur, j - 1]],
                        sem_s).wait()
                pltpu.async_copy(rowb.at[par], agg_sh.at[dsts.at[cur, j]],
                                 sem_s, add=True)
                if j + 1 < SLAB:
                    pltpu.async_copy(xw_hbm.at[gidxs.at[cur, j + 1]],
                                     rowb.at[1 - par], sem)
                else:
                    @pl.when(sl + 1 < NSLAB)
                    def _():
                        pltpu.async_copy(xw_hbm.at[gidxs.at[nxt, 0]],
                                         rowb.at[1 - par], sem)
            return 0
        lax.fori_loop(0, NSLAB, _slab, 0)

        pltpu.make_async_copy(
            rowb.at[1], agg_sh.at[dsts.at[(NSLAB - 1) % 2, SLAB - 1]],
            sem_s).wait()

        plsc.subcore_barrier()

        @pl.when(s == 0)
        def _():
            pltpu.sync_copy(agg_sh.at[pl.ds(0, N)], agg_out.at[c])

    return pl.kernel(body, out_type=out_type, mesh=mesh,
                     scratch_types=scratch)


_make_sc_agg = functools.lru_cache(maxsize=None)(_make_sc_agg)


def _sc_agg(*args):
    return _make_sc_agg()(*args)


def _make_sc_deg():
    mesh = plsc.VectorSubcoreMesh(core_axis_name="c", subcore_axis_name="s")

    out_type = jax.ShapeDtypeStruct((NC, N, D), jnp.float32)

    scratch = [
        pltpu.VMEM((2, SLAB, CH), jnp.int32),
        pltpu.VMEM((CH, D), jnp.float32),
        pltpu.SemaphoreType.DMA,
        pltpu.VMEM_SHARED((N, D), jnp.float32),
    ]

    def body(dst_hbm, z_hbm, ones_hbm, deg_out, dsts, onesb, sem_s, deg_sh):
        c = lax.axis_index("c")
        s = lax.axis_index("s")
        w = c * NS + s

        @pl.when(s == 0)
        def _():
            pltpu.sync_copy(z_hbm, deg_sh)
        pltpu.sync_copy(ones_hbm, onesb)

        plsc.subcore_barrier()

        pltpu.sync_copy(dst_hbm.at[w, 0], dsts.at[0])

        def _slab(sl, _):
            cur = lax.rem(sl, 2)
            nxt = lax.rem(sl + 1, 2)

            @pl.when(sl > 0)
            def _():
                for j in range(SLAB):
                    pltpu.make_async_copy(
                        onesb, deg_sh.at[dsts.at[nxt, j]], sem_s).wait()

            @pl.when(sl + 1 < NSLAB)
            def _():
                pltpu.sync_copy(dst_hbm.at[w, sl + 1], dsts.at[nxt])

            for j in range(SLAB):
                pltpu.async_copy(onesb, deg_sh.at[dsts.at[cur, j]],
                                 sem_s, add=True)
            return 0
        lax.fori_loop(0, NSLAB, _slab, 0)

        for j in range(SLAB):
            pltpu.make_async_copy(
                onesb, deg_sh.at[dsts.at[(NSLAB - 1) % 2, j]], sem_s).wait()

        plsc.subcore_barrier()

        @pl.when(s == 0)
        def _():
            pltpu.sync_copy(deg_sh, deg_out.at[c])

    return pl.kernel(body, out_type=out_type, mesh=mesh,
                     scratch_types=scratch)


_make_sc_deg = functools.lru_cache(maxsize=None)(_make_sc_deg)


def _sc_deg(*args):
    return _make_sc_deg()(*args)


def _comb_body(hs_ref, a_ref, d_ref, b_ref, g_ref, be_ref, o_ref):
    a = a_ref[0] + a_ref[1]
    dsum = d_ref[0, :, 0] + d_ref[1, :, 0]
    rd = 1.0 / jnp.maximum(dsum, 1.0)
    h = hs_ref[...] + a * rd[:, None] + b_ref[0]
    h = jnp.maximum(h, 0.0)
    mu = jnp.mean(h, axis=1, keepdims=True)
    var = jnp.mean((h - mu) ** 2, axis=1, keepdims=True)
    o_ref[...] = (h - mu) / jnp.sqrt(var + EPS) * g_ref[0] + be_ref[0]


def _combine(hself, agg2, degs, b, g, be):
    return pl.pallas_call(
        _comb_body,
        grid=(NBC,),
        in_specs=[
            pl.BlockSpec((BNC, D), lambda nb: (nb, 0)),
            pl.BlockSpec((NC, BNC, D), lambda nb: (0, nb, 0)),
            pl.BlockSpec((NC, BNC, D), lambda nb: (0, nb, 0)),
            pl.BlockSpec((1, D), lambda nb: (0, 0)),
            pl.BlockSpec((1, D), lambda nb: (0, 0)),
            pl.BlockSpec((1, D), lambda nb: (0, 0)),
        ],
        out_specs=pl.BlockSpec((BNC, D), lambda nb: (nb, 0)),
        out_shape=jax.ShapeDtypeStruct((N, D), jnp.float32),
    )(hself, agg2, degs, b.reshape(1, D), g.reshape(1, D), be.reshape(1, D))


def kernel(node_features, edge_index, edge_types,
           W_rel1, W_self1, b1, g1, be1,
           W_rel2, W_self2, b2, g2, be2):
    gidx4 = (edge_types * N + edge_index[0]).reshape(NW, NSLAB, SLAB, CH)
    dst4 = edge_index[1].reshape(NW, NSLAB, SLAB, CH)
    zros = jnp.zeros((NA, D), jnp.float32)
    ones = jnp.ones((CH, D), jnp.float32)

    degs = _sc_deg(dst4, zros, ones)

    w_all1 = jnp.concatenate([W_rel1, W_self1[None]], axis=0)
    xw1 = _xw(node_features, w_all1)
    agg1 = _sc_agg(xw1.reshape((R + 1) * N, D), gidx4, dst4, zros)
    h1 = _combine(xw1[R], agg1, degs, b1, g1, be1)

    w_all2 = jnp.concatenate([W_rel2, W_self2[None]], axis=0)
    xw2 = _xw(h1, w_all2)
    agg2 = _sc_agg(xw2.reshape((R + 1) * N, D), gidx4, dst4, zros)
    h2 = _combine(xw2[R], agg2, degs, b2, g2, be2)
    return h2
